# baseline (speedup 1.0000x reference)
import math

import jax
import jax.numpy as jnp
from jax import lax
from jax.experimental import pallas as pl
from jax.experimental.pallas import tpu as pltpu

_EXP2_SCALE = 8.0 * math.log2(math.e)


def _sdpa_body(q_ref, k_ref, v_ref, o_ref, vext_ref):
    qk = lax.dot_general(
        q_ref[...], k_ref[...],
        dimension_numbers=(((2,), (2,)), ((0,), (0,))),
        preferred_element_type=jnp.float32)

    m = jnp.max(qk, axis=-1, keepdims=True)
    xb = (qk - m).astype(jnp.bfloat16)
    p = jnp.exp2(xb * jnp.bfloat16(_EXP2_SCALE))

    bb, s, d = q_ref.shape
    vext_ref[..., 0:d] = v_ref[...]
    vext_ref[..., d:d + 1] = jnp.ones((bb, s, 1), jnp.bfloat16)

    pv = lax.dot_general(
        p, vext_ref[...],
        dimension_numbers=(((2,), (1,)), ((0,), (0,))),
        preferred_element_type=jnp.float32)

    denom = pv[..., d:d + 1]
    o_ref[...] = (pv[..., 0:d] * (1.0 / denom)).astype(jnp.bfloat16)


def kernel(query, key, value):
    B, S, D = query.shape
    block_b = 8
    grid = (B // block_b,)

    q = query.astype(jnp.bfloat16)
    k = key.astype(jnp.bfloat16)
    v = value.astype(jnp.bfloat16)

    spec = pl.BlockSpec((block_b, S, D), lambda b: (b, 0, 0))
    y = pl.pallas_call(
        _sdpa_body,
        out_shape=jax.ShapeDtypeStruct((B, S, D), jnp.bfloat16),
        grid=grid,
        in_specs=[spec, spec, spec],
        out_specs=spec,
        scratch_shapes=[pltpu.VMEM((block_b, S, 128), jnp.bfloat16)],
        compiler_params=pltpu.CompilerParams(
            dimension_semantics=("parallel",),
            vmem_limit_bytes=25 * 1024 * 1024),
    )(q, k, v)

    return y.astype(jnp.float32)
